# Initial kernel scaffold; baseline (speedup 1.0000x reference)
import jax
import jax.numpy as jnp
from jax.experimental import pallas as pl


def kernel(x, emb_table):
    raise NotImplementedError("write your pallas kernel here")


# speedup vs baseline: 2.1002x; 2.1002x over previous
import functools

import jax
import jax.numpy as jnp
import numpy as np
from jax import lax
from jax.experimental import pallas as pl
from jax.experimental.pallas import tpu as pltpu
from jax.experimental.pallas import tpu_sc as plsc

VOCAB = 100000
D_MODEL = 768
SEQ_LEN = 4096
BATCH = 4

NUM_CORES = 2
NUM_SUBCORES = 16
NUM_WORKERS = NUM_CORES * NUM_SUBCORES

TOKENS = BATCH * SEQ_LEN
TOK_PER_W = TOKENS // NUM_WORKERS
CHUNK = 64
N_CHUNKS = TOK_PER_W // CHUNK
LANES = 16
D_VECS = D_MODEL // LANES


def _pos_encoding_np(max_len: int, d_model: int) -> np.ndarray:
    pos = np.arange(max_len, dtype=np.float32)[:, None]
    _2i = np.arange(0, d_model, 2, dtype=np.float32)
    enc = np.zeros((max_len, d_model), dtype=np.float32)
    angle = pos / np.power(np.float32(10000.0), _2i / np.float32(d_model))
    enc[:, 0::2] = np.sin(angle)
    enc[:, 1::2] = np.cos(angle)
    return enc


_POS_ENC = _pos_encoding_np(SEQ_LEN, D_MODEL)


def _sc_body(x_hbm, pos_hbm, table_hbm, out_hbm, idx_v, buf, pos_v, sem):
    wid = lax.axis_index("s") * NUM_CORES + lax.axis_index("c")
    base = wid * TOK_PER_W
    pos_base = lax.rem(base, SEQ_LEN)

    pltpu.sync_copy(x_hbm.at[pl.ds(wid * N_CHUNKS, N_CHUNKS)], idx_v)

    for j in range(N_CHUNKS):
        gather = pltpu.async_copy(table_hbm.at[idx_v.at[j]], buf, sem)
        pltpu.sync_copy(pos_hbm.at[pl.ds(pos_base + j * CHUNK, CHUNK)], pos_v)
        gather.wait()

        def add_row(r, _):
            for c in range(D_VECS):
                sl = pl.ds(c * LANES, LANES)
                buf[r, sl] = buf[r, sl] + pos_v[r, sl]
            return 0

        lax.fori_loop(0, CHUNK, add_row, 0)
        pltpu.sync_copy(buf, out_hbm.at[pl.ds(base + j * CHUNK, CHUNK)])


@functools.partial(jax.jit, static_argnames=())
def _embed(x_flat, emb_table, pos_enc):
    mesh = plsc.VectorSubcoreMesh(core_axis_name="c", subcore_axis_name="s")
    run = pl.kernel(
        _sc_body,
        out_type=jax.ShapeDtypeStruct((TOKENS, D_MODEL), jnp.float32),
        mesh=mesh,
        scratch_types=[
            pltpu.VMEM((N_CHUNKS, CHUNK), jnp.int32),
            pltpu.VMEM((CHUNK, D_MODEL), jnp.float32),
            pltpu.VMEM((CHUNK, D_MODEL), jnp.float32),
            pltpu.SemaphoreType.DMA,
        ],
    )
    return run(x_flat, pos_enc, emb_table)


def kernel(x, emb_table):
    x_flat = x.reshape(TOKENS // CHUNK, CHUNK).astype(jnp.int32)
    pos_enc = jnp.asarray(_POS_ENC)
    out = _embed(x_flat, emb_table, pos_enc)
    return out.reshape(BATCH, SEQ_LEN, D_MODEL)
